# baseline (speedup 1.0000x reference)
import jax
import jax.numpy as jnp
from jax import lax
from jax.experimental import pallas as pl
from jax.experimental.pallas import tpu as pltpu
from jax.experimental.pallas import tpu_sc as plsc

_BATCH = 1024
_ND = 100
_NS = 20
_NCH = 1000
_ROW = _ND + _ND * _NS
_NCORES = 2
_NSUB = 16
_NW = _NCORES * _NSUB
_L = 16
_BPW = _BATCH // _NW
_NPASS = 4
_CPP = _NCH // _NPASS
_NCHUNK = 5
_CROWS = _CPP // _NCHUNK


def _body(xt_hbm, out_hbm, xblk_v, idx_v, dense_v, *sems):
    cid = lax.axis_index("c")
    sid = lax.axis_index("s")
    wid = sid * _NCORES + cid

    lane = lax.iota(jnp.int32, _L)
    zeros = jnp.zeros((_L,), jnp.float32)
    zero_i = jnp.zeros((_L,), jnp.int32)

    pending = [None] * _NCHUNK
    for g in range(2):
        b0 = (wid * _BPW + g * _L).astype(jnp.int32)
        pltpu.sync_copy(xt_hbm.at[:, pl.ds(b0, _L)], xblk_v)

        def _cvt(d, carry):
            idx_v[d, :] = xblk_v[d, :].astype(jnp.int32)
            return carry
        lax.fori_loop(0, _ND, _cvt, 0)

        for p in range(_NPASS):
            for c in range(_NCHUNK):
                if pending[c] is not None:
                    pending[c].wait()
                    pending[c] = None

                def _z(i, carry):
                    for k in range(_NS):
                        dense_v[c * _CROWS + i, k, 0, :] = zeros
                    return carry
                lax.fori_loop(0, _CROWS, _z, 0, unroll=2)

            def _scan(d, carry):
                ch = idx_v[d, :] - (p * _CPP)
                m = (ch >= 0) & (ch < _CPP)
                for s in range(_NS):
                    v = xblk_v[_ND + d * _NS + s, :]
                    plsc.addupdate_scatter(
                        dense_v, [ch, zero_i + s, zero_i, lane], v, mask=m)
                return carry
            lax.fori_loop(0, _ND, _scan, 0, unroll=2)

            for c in range(_NCHUNK):
                pending[c] = pltpu.async_copy(
                    dense_v.at[pl.ds(c * _CROWS, _CROWS)],
                    out_hbm.at[pl.ds(p * _CPP + c * _CROWS, _CROWS),
                               :, :, pl.ds(b0, _L)],
                    sems[c])
    for c in range(_NCHUNK):
        if pending[c] is not None:
            pending[c].wait()


def kernel(inputs):
    xt = inputs.T
    mesh = plsc.VectorSubcoreMesh(
        core_axis_name="c", subcore_axis_name="s",
        num_cores=_NCORES, num_subcores=_NSUB)
    run = pl.kernel(
        _body,
        out_type=jax.ShapeDtypeStruct((_NCH, _NS, 1, _BATCH), jnp.float32),
        mesh=mesh,
        compiler_params=pltpu.CompilerParams(
            use_tc_tiling_on_sc=False, needs_layout_passes=False),
        scratch_types=[
            pltpu.VMEM((_ROW, _L), jnp.float32),
            pltpu.VMEM((_ND, _L), jnp.int32),
            pltpu.VMEM((_CPP, _NS, 1, _L), jnp.float32),
        ] + [pltpu.SemaphoreType.DMA] * _NCHUNK,
    )
    out = run(xt)
    return out.transpose(3, 0, 1, 2)


# speedup vs baseline: 1.0424x; 1.0424x over previous
import jax
import jax.numpy as jnp
from jax import lax
from jax.experimental import pallas as pl
from jax.experimental.pallas import tpu as pltpu
from jax.experimental.pallas import tpu_sc as plsc

_BATCH = 1024
_ND = 100
_NS = 20
_NCH = 1000
_ROW = _ND + _ND * _NS
_NCORES = 2
_NSUB = 16
_NW = _NCORES * _NSUB
_L = 16
_BPW = _BATCH // _NW
_NPASS = 4
_CPP = _NCH // _NPASS
_NCHUNK = 10
_CROWS = _CPP // _NCHUNK


def _body(xt_hbm, out_hbm, xblk_v, idx_v, dense_v, *sems):
    cid = lax.axis_index("c")
    sid = lax.axis_index("s")
    wid = sid * _NCORES + cid

    lane = lax.iota(jnp.int32, _L)
    zeros = jnp.zeros((_L,), jnp.float32)
    zero_i = jnp.zeros((_L,), jnp.int32)

    pending = [None] * _NCHUNK
    for g in range(2):
        b0 = (wid * _BPW + g * _L).astype(jnp.int32)
        pltpu.sync_copy(xt_hbm.at[:, pl.ds(b0, _L)], xblk_v)

        def _cvt(d, carry):
            idx_v[d, :] = xblk_v[d, :].astype(jnp.int32)
            return carry
        lax.fori_loop(0, _ND, _cvt, 0)

        for p in range(_NPASS):
            for c in range(_NCHUNK):
                if pending[c] is not None:
                    pending[c].wait()
                    pending[c] = None

                def _z(i, carry):
                    for k in range(_NS):
                        dense_v[c * _CROWS + i, k, 0, :] = zeros
                    return carry
                lax.fori_loop(0, _CROWS, _z, 0)

            def _scan(d, carry):
                ch = idx_v[d, :] - (p * _CPP)
                m = (ch >= 0) & (ch < _CPP)
                for s in range(_NS):
                    v = xblk_v[_ND + d * _NS + s, :]
                    plsc.addupdate_scatter(
                        dense_v, [ch, zero_i + s, zero_i, lane], v, mask=m)
                return carry
            lax.fori_loop(0, _ND, _scan, 0)

            for c in range(_NCHUNK):
                pending[c] = pltpu.async_copy(
                    dense_v.at[pl.ds(c * _CROWS, _CROWS)],
                    out_hbm.at[pl.ds(p * _CPP + c * _CROWS, _CROWS),
                               :, :, pl.ds(b0, _L)],
                    sems[c])
    for c in range(_NCHUNK):
        if pending[c] is not None:
            pending[c].wait()


def kernel(inputs):
    xt = inputs.T
    mesh = plsc.VectorSubcoreMesh(
        core_axis_name="c", subcore_axis_name="s",
        num_cores=_NCORES, num_subcores=_NSUB)
    run = pl.kernel(
        _body,
        out_type=jax.ShapeDtypeStruct((_NCH, _NS, 1, _BATCH), jnp.float32),
        mesh=mesh,
        compiler_params=pltpu.CompilerParams(
            use_tc_tiling_on_sc=False, needs_layout_passes=False),
        scratch_types=[
            pltpu.VMEM((_ROW, _L), jnp.float32),
            pltpu.VMEM((_ND, _L), jnp.int32),
            pltpu.VMEM((_CPP, _NS, 1, _L), jnp.float32),
        ] + [pltpu.SemaphoreType.DMA] * _NCHUNK,
    )
    out = run(xt)
    return out.transpose(3, 0, 1, 2)
